# Initial kernel scaffold; baseline (speedup 1.0000x reference)
import functools

import jax
import jax.numpy as jnp
from jax import lax
from jax.experimental import pallas as pl
from jax.experimental.pallas import tpu as pltpu
from jax.experimental.pallas import tpu_sc as plsc

F32 = jnp.float32
_BLK = 1000
_NPAD = 10240
_EB = 80


def _leaky(v):
    return jnp.where(v > 0, v, 0.2 * v)


def _layer_pre(xb, Ws, Wd, Wl, asv, adv, bl, i, nsteps,
               xs_ref, hlin_ref, asrc_ref, adst_ref, m_ref, sm):
    xs = jnp.dot(xb, Ws, preferred_element_type=F32)
    xs_ref[...] = xs
    hlin_ref[...] = jnp.dot(xb, Wl, preferred_element_type=F32) + bl
    a_s = jnp.dot(xs, asv, preferred_element_type=F32)
    asrc_ref[...] = a_s
    wda = jnp.dot(Wd, adv, preferred_element_type=F32)
    a_d = jnp.dot(xb, wda, preferred_element_type=F32)
    adst_ref[...] = a_d
    bs = jnp.max(a_s)
    bd = jnp.max(a_d)

    @pl.when(i == 0)
    def _():
        sm[0] = bs
        sm[1] = bd

    @pl.when(i != 0)
    def _():
        sm[0] = jnp.maximum(sm[0], bs)
        sm[1] = jnp.maximum(sm[1], bd)

    @pl.when(i == nsteps - 1)
    def _():
        m_ref[...] = jnp.full((1, 128), _leaky(sm[0] + sm[1]), F32)


def _pre_body(x_ref, Ws_ref, Wd_ref, Wl_ref, asv_ref, adv_ref, bl_ref,
              xs_ref, hlin_ref, asrc_ref, adst_ref, m_ref, sm):
    i = pl.program_id(0)
    _layer_pre(x_ref[...], Ws_ref[...], Wd_ref[...], Wl_ref[...],
               asv_ref[...], adv_ref[...], bl_ref[...], i, pl.num_programs(0),
               xs_ref, hlin_ref, asrc_ref, adst_ref, m_ref, sm)


def _comb_body(acc_ref, dn_ref, hlin1_ref, b_ref,
               Ws_ref, Wd_ref, Wl_ref, asv_ref, adv_ref, bl_ref,
               xs_ref, hlin_ref, asrc_ref, adst_ref, m_ref, sm):
    i = pl.program_id(0)
    a = acc_ref[0] + acc_ref[1]
    den = jnp.sum(dn_ref[...], axis=1, keepdims=True)
    h = jnp.maximum(a / (den + 1e-16) + b_ref[...] + hlin1_ref[...], 0.0)
    _layer_pre(h, Ws_ref[...], Wd_ref[...], Wl_ref[...],
               asv_ref[...], adv_ref[...], bl_ref[...], i, pl.num_programs(0),
               xs_ref, hlin_ref, asrc_ref, adst_ref, m_ref, sm)


def _final_body(acc_ref, dn_ref, hlin_ref, b_ref, out_ref):
    a = acc_ref[0] + acc_ref[1]
    den = jnp.sum(dn_ref[...], axis=1, keepdims=True)
    out_ref[...] = a / (den + 1e-16) + b_ref[...] + hlin_ref[...]


def _w_spec():
    return pl.BlockSpec((128, 128), lambda i: (0, 0))


def _layer_out(n):
    out_shape = [jax.ShapeDtypeStruct((n, 128), F32),
                 jax.ShapeDtypeStruct((n, 128), F32),
                 jax.ShapeDtypeStruct((n, 1), F32),
                 jax.ShapeDtypeStruct((n, 1), F32),
                 jax.ShapeDtypeStruct((1, 128), F32)]
    out_specs = [pl.BlockSpec((_BLK, 128), lambda i: (i, 0)),
                 pl.BlockSpec((_BLK, 128), lambda i: (i, 0)),
                 pl.BlockSpec((_BLK, 1), lambda i: (i, 0)),
                 pl.BlockSpec((_BLK, 1), lambda i: (i, 0)),
                 pl.BlockSpec((1, 128), lambda i: (0, 0))]
    return out_shape, out_specs


def _precompute(x, Ws, Wd, a_s, a_d, Wl, bl):
    n = x.shape[0]
    out_shape, out_specs = _layer_out(n)
    return pl.pallas_call(
        _pre_body,
        grid=(n // _BLK,),
        in_specs=[pl.BlockSpec((_BLK, 128), lambda i: (i, 0)),
                  _w_spec(), _w_spec(), _w_spec(),
                  pl.BlockSpec((128, 1), lambda i: (0, 0)),
                  pl.BlockSpec((128, 1), lambda i: (0, 0)),
                  pl.BlockSpec((1, 128), lambda i: (0, 0))],
        out_specs=out_specs,
        out_shape=out_shape,
        scratch_shapes=[pltpu.SMEM((2,), F32)],
    )(x, Ws, Wd, Wl, a_s.reshape(128, 1), a_d.reshape(128, 1),
      bl.reshape(1, 128))


def _combine_pre(acc, dnt, hlin1, b, Ws, Wd, a_s, a_d, Wl, bl):
    n = hlin1.shape[0]
    out_shape, out_specs = _layer_out(n)
    return pl.pallas_call(
        _comb_body,
        grid=(n // _BLK,),
        in_specs=[pl.BlockSpec((2, _BLK, 128), lambda i: (0, i, 0)),
                  pl.BlockSpec((_BLK, 2), lambda i: (i, 0)),
                  pl.BlockSpec((_BLK, 128), lambda i: (i, 0)),
                  pl.BlockSpec((1, 128), lambda i: (0, 0)),
                  _w_spec(), _w_spec(), _w_spec(),
                  pl.BlockSpec((128, 1), lambda i: (0, 0)),
                  pl.BlockSpec((128, 1), lambda i: (0, 0)),
                  pl.BlockSpec((1, 128), lambda i: (0, 0))],
        out_specs=out_specs,
        out_shape=out_shape,
        scratch_shapes=[pltpu.SMEM((2,), F32)],
    )(acc, dnt, hlin1, b.reshape(1, 128), Ws, Wd, Wl,
      a_s.reshape(128, 1), a_d.reshape(128, 1), bl.reshape(1, 128))


def _final(acc, dnt, hlin, b):
    n = hlin.shape[0]
    return pl.pallas_call(
        _final_body,
        grid=(n // _BLK,),
        in_specs=[pl.BlockSpec((2, _BLK, 128), lambda i: (0, i, 0)),
                  pl.BlockSpec((_BLK, 2), lambda i: (i, 0)),
                  pl.BlockSpec((_BLK, 128), lambda i: (i, 0)),
                  pl.BlockSpec((1, 128), lambda i: (0, 0))],
        out_specs=pl.BlockSpec((_BLK, 128), lambda i: (i, 0)),
        out_shape=jax.ShapeDtypeStruct((n, 128), F32),
    )(acc, dnt, hlin, b.reshape(1, 128))


def _gat_sc(src, dst, xs, asrc, adst, m):
    n = xs.shape[0]
    e = src.shape[0]
    ep = e // 32
    nchunk = ep // _EB
    assert e % 32 == 0 and ep % _EB == 0
    rows_per_tile = _NPAD // 16

    mesh = plsc.VectorSubcoreMesh(core_axis_name="c", subcore_axis_name="s")
    z16 = jnp.zeros((16,), F32)

    def body(src_h, dst_h, xs_h, asrc_h, adst_h, m_h,
             acc_h, den_h,
             src_t, dst_t, asrc_t, adst_t, m_t,
             srcb, dstb, pbuf, rows, zbuf, acc_s, den_s):
        cid = lax.axis_index("c")
        sid = lax.axis_index("s")
        wid = cid * 16 + sid
        base = wid * ep
        pltpu.sync_copy(src_h.at[pl.ds(base, ep)], src_t)
        pltpu.sync_copy(dst_h.at[pl.ds(base, ep)], dst_t)
        pltpu.sync_copy(asrc_h, asrc_t)
        pltpu.sync_copy(adst_h, adst_t)
        pltpu.sync_copy(m_h.at[pl.ds(0, 16)], m_t)

        @pl.loop(0, _EB)
        def _(r):
            for k in range(8):
                rows[r, pl.ds(k * 16, 16)] = z16

        @pl.loop(0, rows_per_tile, step=16)
        def _(i):
            zbuf[pl.ds(i, 16)] = z16

        off = sid * rows_per_tile
        for j in range(rows_per_tile // _EB):
            pltpu.sync_copy(rows, acc_s.at[pl.ds(off + j * _EB, _EB)])
        pltpu.sync_copy(zbuf, den_s.at[pl.ds(off, rows_per_tile)])
        plsc.subcore_barrier()

        m_v = m_t[...]

        @pl.loop(0, nchunk)
        def _(c):
            cb = c * _EB
            for k in range(_EB // 16):
                sv = src_t[pl.ds(cb + k * 16, 16)]
                dv = dst_t[pl.ds(cb + k * 16, 16)]
                srcb[pl.ds(k * 16, 16)] = sv
                dstb[pl.ds(k * 16, 16)] = dv
                ev = (plsc.load_gather(asrc_t, [sv])
                      + plsc.load_gather(adst_t, [dv]))
                ev = jnp.where(ev > 0, ev, ev * 0.2)
                pbuf[pl.ds(k * 16, 16)] = jnp.exp(ev - m_v)
            pltpu.sync_copy(xs_h.at[srcb], rows)

            @pl.loop(0, _EB)
            def _(r):
                pv = lax.broadcast(pbuf[r], (16,))
                for k in range(8):
                    rows[r, pl.ds(k * 16, 16)] = rows[r, pl.ds(k * 16, 16)] * pv

            pltpu.sync_copy(rows, acc_s.at[dstb], add=True)
            pltpu.sync_copy(pbuf, den_s.at[dstb], add=True)

        plsc.subcore_barrier()
        for j in range(rows_per_tile // _EB):
            o = off + j * _EB
            pltpu.sync_copy(acc_s.at[pl.ds(o, _EB)],
                            acc_h.at[cid, pl.ds(o, _EB)])
        pltpu.sync_copy(den_s.at[pl.ds(off, rows_per_tile)],
                        den_h.at[cid, pl.ds(off, rows_per_tile)])

    k = pl.kernel(
        body,
        out_type=[jax.ShapeDtypeStruct((2, _NPAD, 128), F32),
                  jax.ShapeDtypeStruct((2, _NPAD), F32)],
        mesh=mesh,
        scratch_types=[
            pltpu.VMEM((ep,), jnp.int32),
            pltpu.VMEM((ep,), jnp.int32),
            pltpu.VMEM((n,), F32),
            pltpu.VMEM((n,), F32),
            pltpu.VMEM((16,), F32),
            pltpu.VMEM((_EB,), jnp.int32),
            pltpu.VMEM((_EB,), jnp.int32),
            pltpu.VMEM((_EB,), F32),
            pltpu.VMEM((_EB, 128), F32),
            pltpu.VMEM((rows_per_tile,), F32),
            pltpu.VMEM_SHARED((_NPAD, 128), F32),
            pltpu.VMEM_SHARED((_NPAD,), F32),
        ],
    )
    return k(src, dst, xs, asrc, adst, m)


def kernel(x, edge_index, Ws1, Wd1, as1, ad1, b1, Wl1, bl1,
           Ws2, Wd2, as2, ad2, b2, Wl2, bl2):
    n = x.shape[0]
    src = edge_index[0]
    dst = edge_index[1]

    xs1, hlin1, asrc1, adst1, m1 = _precompute(x, Ws1, Wd1, as1, ad1, Wl1, bl1)
    acc1, den1 = _gat_sc(src, dst, xs1, asrc1.reshape(n), adst1.reshape(n),
                         m1.reshape(128))
    xs2, hlin2, asrc2, adst2, m2 = _combine_pre(
        acc1, den1.T, hlin1, b1, Ws2, Wd2, as2, ad2, Wl2, bl2)
    acc2, den2 = _gat_sc(src, dst, xs2, asrc2.reshape(n), adst2.reshape(n),
                         m2.reshape(128))
    return _final(acc2, den2.T, hlin2, b2)


# speedup vs baseline: 27.1648x; 27.1648x over previous
import dataclasses
import functools

import jax
import jax.numpy as jnp
from jax import lax
from jax.experimental import pallas as pl
from jax.experimental.pallas import tpu as pltpu
from jax.experimental.pallas import tpu_sc as plsc

F32 = jnp.float32
_BLK = 1000
_NPAD = 10240
_EB = 80


def _leaky(v):
    return jnp.where(v > 0, v, 0.2 * v)


def _layer_pre(xb, Ws, Wd, Wl, asv, adv, bl, i, nsteps,
               xs_ref, hlin_ref, asrc_ref, adst_ref, m_ref, sm):
    xs = jnp.dot(xb, Ws, preferred_element_type=F32)
    xs_ref[...] = xs
    hlin_ref[...] = jnp.dot(xb, Wl, preferred_element_type=F32) + bl
    a_s = jnp.dot(xs, asv, preferred_element_type=F32)
    asrc_ref[...] = a_s
    wda = jnp.dot(Wd, adv, preferred_element_type=F32)
    a_d = jnp.dot(xb, wda, preferred_element_type=F32)
    adst_ref[...] = a_d
    bs = jnp.max(a_s)
    bd = jnp.max(a_d)

    @pl.when(i == 0)
    def _():
        sm[0] = bs
        sm[1] = bd

    @pl.when(i != 0)
    def _():
        sm[0] = jnp.maximum(sm[0], bs)
        sm[1] = jnp.maximum(sm[1], bd)

    @pl.when(i == nsteps - 1)
    def _():
        m_ref[...] = jnp.full((1, 128), _leaky(sm[0] + sm[1]), F32)


def _pre_body(x_ref, Ws_ref, Wd_ref, Wl_ref, asv_ref, adv_ref, bl_ref,
              xs_ref, hlin_ref, asrc_ref, adst_ref, m_ref, sm):
    i = pl.program_id(0)
    _layer_pre(x_ref[...], Ws_ref[...], Wd_ref[...], Wl_ref[...],
               asv_ref[...], adv_ref[...], bl_ref[...], i, pl.num_programs(0),
               xs_ref, hlin_ref, asrc_ref, adst_ref, m_ref, sm)


def _comb_body(acc_ref, dn_ref, hlin1_ref, b_ref,
               Ws_ref, Wd_ref, Wl_ref, asv_ref, adv_ref, bl_ref,
               xs_ref, hlin_ref, asrc_ref, adst_ref, m_ref, sm):
    i = pl.program_id(0)
    a = acc_ref[0] + acc_ref[1]
    den = jnp.sum(dn_ref[...], axis=1, keepdims=True)
    h = jnp.maximum(a / (den + 1e-16) + b_ref[...] + hlin1_ref[...], 0.0)
    _layer_pre(h, Ws_ref[...], Wd_ref[...], Wl_ref[...],
               asv_ref[...], adv_ref[...], bl_ref[...], i, pl.num_programs(0),
               xs_ref, hlin_ref, asrc_ref, adst_ref, m_ref, sm)


def _final_body(acc_ref, dn_ref, hlin_ref, b_ref, out_ref):
    a = acc_ref[0] + acc_ref[1]
    den = jnp.sum(dn_ref[...], axis=1, keepdims=True)
    out_ref[...] = a / (den + 1e-16) + b_ref[...] + hlin_ref[...]


def _w_spec():
    return pl.BlockSpec((128, 128), lambda i: (0, 0))


def _layer_out(n):
    out_shape = [jax.ShapeDtypeStruct((n, 128), F32),
                 jax.ShapeDtypeStruct((n, 128), F32),
                 jax.ShapeDtypeStruct((n, 1), F32),
                 jax.ShapeDtypeStruct((n, 1), F32),
                 jax.ShapeDtypeStruct((1, 128), F32)]
    out_specs = [pl.BlockSpec((_BLK, 128), lambda i: (i, 0)),
                 pl.BlockSpec((_BLK, 128), lambda i: (i, 0)),
                 pl.BlockSpec((_BLK, 1), lambda i: (i, 0)),
                 pl.BlockSpec((_BLK, 1), lambda i: (i, 0)),
                 pl.BlockSpec((1, 128), lambda i: (0, 0))]
    return out_shape, out_specs


def _precompute(x, Ws, Wd, a_s, a_d, Wl, bl):
    n = x.shape[0]
    out_shape, out_specs = _layer_out(n)
    return pl.pallas_call(
        _pre_body,
        grid=(n // _BLK,),
        in_specs=[pl.BlockSpec((_BLK, 128), lambda i: (i, 0)),
                  _w_spec(), _w_spec(), _w_spec(),
                  pl.BlockSpec((128, 1), lambda i: (0, 0)),
                  pl.BlockSpec((128, 1), lambda i: (0, 0)),
                  pl.BlockSpec((1, 128), lambda i: (0, 0))],
        out_specs=out_specs,
        out_shape=out_shape,
        scratch_shapes=[pltpu.SMEM((2,), F32)],
    )(x, Ws, Wd, Wl, a_s.reshape(128, 1), a_d.reshape(128, 1),
      bl.reshape(1, 128))


def _combine_pre(acc, dnt, hlin1, b, Ws, Wd, a_s, a_d, Wl, bl):
    n = hlin1.shape[0]
    out_shape, out_specs = _layer_out(n)
    return pl.pallas_call(
        _comb_body,
        grid=(n // _BLK,),
        in_specs=[pl.BlockSpec((2, _BLK, 128), lambda i: (0, i, 0)),
                  pl.BlockSpec((_BLK, 2), lambda i: (i, 0)),
                  pl.BlockSpec((_BLK, 128), lambda i: (i, 0)),
                  pl.BlockSpec((1, 128), lambda i: (0, 0)),
                  _w_spec(), _w_spec(), _w_spec(),
                  pl.BlockSpec((128, 1), lambda i: (0, 0)),
                  pl.BlockSpec((128, 1), lambda i: (0, 0)),
                  pl.BlockSpec((1, 128), lambda i: (0, 0))],
        out_specs=out_specs,
        out_shape=out_shape,
        scratch_shapes=[pltpu.SMEM((2,), F32)],
    )(acc, dnt, hlin1, b.reshape(1, 128), Ws, Wd, Wl,
      a_s.reshape(128, 1), a_d.reshape(128, 1), bl.reshape(1, 128))


def _final(acc, dnt, hlin, b):
    n = hlin.shape[0]
    return pl.pallas_call(
        _final_body,
        grid=(n // _BLK,),
        in_specs=[pl.BlockSpec((2, _BLK, 128), lambda i: (0, i, 0)),
                  pl.BlockSpec((_BLK, 2), lambda i: (i, 0)),
                  pl.BlockSpec((_BLK, 128), lambda i: (i, 0)),
                  pl.BlockSpec((1, 128), lambda i: (0, 0))],
        out_specs=pl.BlockSpec((_BLK, 128), lambda i: (i, 0)),
        out_shape=jax.ShapeDtypeStruct((n, 128), F32),
    )(acc, dnt, hlin, b.reshape(1, 128))


def _gat_sc(src, dst, xs, asrc, adst, m):
    n = xs.shape[0]
    e = src.shape[0]
    ep = e // 32
    eblk = 2000
    nblk = ep // eblk
    nchunk = eblk // _EB
    assert e % 32 == 0 and ep % eblk == 0 and eblk % _EB == 0
    rows_per_tile = _NPAD // 16

    mesh = plsc.VectorSubcoreMesh(core_axis_name="c", subcore_axis_name="s")

    def body(src_h, dst_h, xs_h, asrc_h, adst_h, m_h,
             acc_h, den_h,
             src_t, dst_t, asrc_t, adst_t, m_t,
             srcb, dstb, pbuf, rows, zbuf, acc_s, den_s):
        cid = lax.axis_index("c")
        sid = lax.axis_index("s")
        wid = cid * 16 + sid
        base = wid * ep
        pltpu.sync_copy(asrc_h, asrc_t)
        pltpu.sync_copy(adst_h, adst_t)
        pltpu.sync_copy(m_h.at[pl.ds(0, 16)], m_t)
        z16 = jnp.zeros((16,), F32)

        @pl.loop(0, _EB)
        def _(r):
            for k in range(8):
                rows[r, pl.ds(k * 16, 16)] = z16

        @pl.loop(0, rows_per_tile, step=16)
        def _(i):
            zbuf[pl.ds(i, 16)] = z16

        off = sid * rows_per_tile
        for j in range(rows_per_tile // _EB):
            pltpu.sync_copy(rows, acc_s.at[pl.ds(off + j * _EB, _EB)])
        pltpu.sync_copy(zbuf, den_s.at[pl.ds(off, rows_per_tile)])
        plsc.subcore_barrier()

        m_v = m_t[...]

        @pl.loop(0, nblk)
        def _(blk):
            bb = base + blk * eblk
            pltpu.sync_copy(src_h.at[pl.ds(bb, eblk)], src_t)
            pltpu.sync_copy(dst_h.at[pl.ds(bb, eblk)], dst_t)

            @pl.loop(0, nchunk)
            def _(c):
                cb = c * _EB
                for k in range(_EB // 16):
                    sv = src_t[pl.ds(cb + k * 16, 16)]
                    dv = dst_t[pl.ds(cb + k * 16, 16)]
                    srcb[pl.ds(k * 16, 16)] = sv
                    dstb[pl.ds(k * 16, 16)] = dv
                    ev = (plsc.load_gather(asrc_t, [sv])
                          + plsc.load_gather(adst_t, [dv]))
                    ev = jnp.where(ev > 0, ev, ev * 0.2)
                    pbuf[pl.ds(k * 16, 16)] = jnp.exp(ev - m_v)
                pltpu.sync_copy(xs_h.at[srcb], rows)

                @pl.loop(0, _EB)
                def _(r):
                    pv = plsc.load_gather(pbuf, [lax.broadcast(r, (16,))])
                    for k in range(8):
                        rows[r, pl.ds(k * 16, 16)] = (
                            rows[r, pl.ds(k * 16, 16)] * pv)

                pltpu.sync_copy(rows, acc_s.at[dstb], add=True)
                pltpu.sync_copy(pbuf, den_s.at[dstb], add=True)

        plsc.subcore_barrier()
        for j in range(rows_per_tile // _EB):
            o = off + j * _EB
            pltpu.sync_copy(acc_s.at[pl.ds(o, _EB)],
                            acc_h.at[cid, pl.ds(o, _EB)])
        pltpu.sync_copy(den_s.at[pl.ds(off, rows_per_tile)],
                        den_h.at[cid, pl.ds(off, rows_per_tile)])

    cp = pltpu.CompilerParams()
    if "needs_layout_passes" in pltpu.CompilerParams.__dataclass_fields__:
        cp = dataclasses.replace(cp, needs_layout_passes=False)
    k = pl.kernel(
        body,
        out_type=[jax.ShapeDtypeStruct((2, _NPAD, 128), F32),
                  jax.ShapeDtypeStruct((2, _NPAD), F32)],
        mesh=mesh,
        compiler_params=cp,
        scratch_types=[
            pltpu.VMEM((eblk,), jnp.int32),
            pltpu.VMEM((eblk,), jnp.int32),
            pltpu.VMEM((n,), F32),
            pltpu.VMEM((n,), F32),
            pltpu.VMEM((16,), F32),
            pltpu.VMEM((_EB,), jnp.int32),
            pltpu.VMEM((_EB,), jnp.int32),
            pltpu.VMEM((_EB,), F32),
            pltpu.VMEM((_EB, 128), F32),
            pltpu.VMEM((rows_per_tile,), F32),
            pltpu.VMEM_SHARED((_NPAD, 128), F32),
            pltpu.VMEM_SHARED((_NPAD,), F32),
        ],
    )
    return k(src, dst, xs, asrc, adst, m)


def kernel(x, edge_index, Ws1, Wd1, as1, ad1, b1, Wl1, bl1,
           Ws2, Wd2, as2, ad2, b2, Wl2, bl2):
    n = x.shape[0]
    src = edge_index[0]
    dst = edge_index[1]

    xs1, hlin1, asrc1, adst1, m1 = _precompute(x, Ws1, Wd1, as1, ad1, Wl1, bl1)
    acc1, den1 = _gat_sc(src, dst, xs1, asrc1.reshape(n), adst1.reshape(n),
                         m1.reshape(128))
    xs2, hlin2, asrc2, adst2, m2 = _combine_pre(
        acc1, den1.T, hlin1, b1, Ws2, Wd2, as2, ad2, Wl2, bl2)
    acc2, den2 = _gat_sc(src, dst, xs2, asrc2.reshape(n), adst2.reshape(n),
                         m2.reshape(128))
    return _final(acc2, den2.T, hlin2, b2)


# speedup vs baseline: 48.9707x; 1.8027x over previous
import dataclasses
import functools

import jax
import jax.numpy as jnp
from jax import lax
from jax.experimental import pallas as pl
from jax.experimental.pallas import tpu as pltpu
from jax.experimental.pallas import tpu_sc as plsc

F32 = jnp.float32
_BLK = 1000
_NPAD = 10240
_EB = 80


def _leaky(v):
    return jnp.where(v > 0, v, 0.2 * v)


def _layer_pre(xb, Ws, Wd, Wl, asv, adv, bl, i, nsteps,
               xs_ref, hlin_ref, asrc_ref, adst_ref, m_ref, sm):
    xs = jnp.dot(xb, Ws, preferred_element_type=F32)
    xs_ref[...] = xs
    hlin_ref[...] = jnp.dot(xb, Wl, preferred_element_type=F32) + bl
    a_s = jnp.dot(xs, asv, preferred_element_type=F32)
    asrc_ref[...] = a_s
    wda = jnp.dot(Wd, adv, preferred_element_type=F32)
    a_d = jnp.dot(xb, wda, preferred_element_type=F32)
    adst_ref[...] = a_d
    bs = jnp.max(a_s)
    bd = jnp.max(a_d)

    @pl.when(i == 0)
    def _():
        sm[0] = bs
        sm[1] = bd

    @pl.when(i != 0)
    def _():
        sm[0] = jnp.maximum(sm[0], bs)
        sm[1] = jnp.maximum(sm[1], bd)

    @pl.when(i == nsteps - 1)
    def _():
        m_ref[...] = jnp.full((1, 128), _leaky(sm[0] + sm[1]), F32)


def _pre_body(x_ref, Ws_ref, Wd_ref, Wl_ref, asv_ref, adv_ref, bl_ref,
              xs_ref, hlin_ref, asrc_ref, adst_ref, m_ref, sm):
    i = pl.program_id(0)
    _layer_pre(x_ref[...], Ws_ref[...], Wd_ref[...], Wl_ref[...],
               asv_ref[...], adv_ref[...], bl_ref[...], i, pl.num_programs(0),
               xs_ref, hlin_ref, asrc_ref, adst_ref, m_ref, sm)


def _comb_body(acc_ref, dn_ref, hlin1_ref, b_ref,
               Ws_ref, Wd_ref, Wl_ref, asv_ref, adv_ref, bl_ref,
               xs_ref, hlin_ref, asrc_ref, adst_ref, m_ref, sm):
    i = pl.program_id(0)
    a = acc_ref[0] + acc_ref[1]
    den = jnp.sum(dn_ref[...], axis=1, keepdims=True)
    h = jnp.maximum(a / (den + 1e-16) + b_ref[...] + hlin1_ref[...], 0.0)
    _layer_pre(h, Ws_ref[...], Wd_ref[...], Wl_ref[...],
               asv_ref[...], adv_ref[...], bl_ref[...], i, pl.num_programs(0),
               xs_ref, hlin_ref, asrc_ref, adst_ref, m_ref, sm)


def _final_body(acc_ref, dn_ref, hlin_ref, b_ref, out_ref):
    a = acc_ref[0] + acc_ref[1]
    den = jnp.sum(dn_ref[...], axis=1, keepdims=True)
    out_ref[...] = a / (den + 1e-16) + b_ref[...] + hlin_ref[...]


def _w_spec():
    return pl.BlockSpec((128, 128), lambda i: (0, 0))


def _layer_out(n):
    out_shape = [jax.ShapeDtypeStruct((n, 128), F32),
                 jax.ShapeDtypeStruct((n, 128), F32),
                 jax.ShapeDtypeStruct((n, 1), F32),
                 jax.ShapeDtypeStruct((n, 1), F32),
                 jax.ShapeDtypeStruct((1, 128), F32)]
    out_specs = [pl.BlockSpec((_BLK, 128), lambda i: (i, 0)),
                 pl.BlockSpec((_BLK, 128), lambda i: (i, 0)),
                 pl.BlockSpec((_BLK, 1), lambda i: (i, 0)),
                 pl.BlockSpec((_BLK, 1), lambda i: (i, 0)),
                 pl.BlockSpec((1, 128), lambda i: (0, 0))]
    return out_shape, out_specs


def _precompute(x, Ws, Wd, a_s, a_d, Wl, bl):
    n = x.shape[0]
    out_shape, out_specs = _layer_out(n)
    return pl.pallas_call(
        _pre_body,
        grid=(n // _BLK,),
        in_specs=[pl.BlockSpec((_BLK, 128), lambda i: (i, 0)),
                  _w_spec(), _w_spec(), _w_spec(),
                  pl.BlockSpec((128, 1), lambda i: (0, 0)),
                  pl.BlockSpec((128, 1), lambda i: (0, 0)),
                  pl.BlockSpec((1, 128), lambda i: (0, 0))],
        out_specs=out_specs,
        out_shape=out_shape,
        scratch_shapes=[pltpu.SMEM((2,), F32)],
    )(x, Ws, Wd, Wl, a_s.reshape(128, 1), a_d.reshape(128, 1),
      bl.reshape(1, 128))


def _combine_pre(acc, dnt, hlin1, b, Ws, Wd, a_s, a_d, Wl, bl):
    n = hlin1.shape[0]
    out_shape, out_specs = _layer_out(n)
    return pl.pallas_call(
        _comb_body,
        grid=(n // _BLK,),
        in_specs=[pl.BlockSpec((2, _BLK, 128), lambda i: (0, i, 0)),
                  pl.BlockSpec((_BLK, 2), lambda i: (i, 0)),
                  pl.BlockSpec((_BLK, 128), lambda i: (i, 0)),
                  pl.BlockSpec((1, 128), lambda i: (0, 0)),
                  _w_spec(), _w_spec(), _w_spec(),
                  pl.BlockSpec((128, 1), lambda i: (0, 0)),
                  pl.BlockSpec((128, 1), lambda i: (0, 0)),
                  pl.BlockSpec((1, 128), lambda i: (0, 0))],
        out_specs=out_specs,
        out_shape=out_shape,
        scratch_shapes=[pltpu.SMEM((2,), F32)],
    )(acc, dnt, hlin1, b.reshape(1, 128), Ws, Wd, Wl,
      a_s.reshape(128, 1), a_d.reshape(128, 1), bl.reshape(1, 128))


def _final(acc, dnt, hlin, b):
    n = hlin.shape[0]
    return pl.pallas_call(
        _final_body,
        grid=(n // _BLK,),
        in_specs=[pl.BlockSpec((2, _BLK, 128), lambda i: (0, i, 0)),
                  pl.BlockSpec((_BLK, 2), lambda i: (i, 0)),
                  pl.BlockSpec((_BLK, 128), lambda i: (i, 0)),
                  pl.BlockSpec((1, 128), lambda i: (0, 0))],
        out_specs=pl.BlockSpec((_BLK, 128), lambda i: (i, 0)),
        out_shape=jax.ShapeDtypeStruct((n, 128), F32),
    )(acc, dnt, hlin, b.reshape(1, 128))


def _gat_sc(src, dst, xs, asrc, adst, m):
    n = xs.shape[0]
    e = src.shape[0]
    ep = e // 32
    eblk = 2000
    nblk = ep // eblk
    nchunk = eblk // _EB
    assert e % 32 == 0 and ep % eblk == 0 and eblk % _EB == 0
    rows_per_tile = _NPAD // 16

    mesh = plsc.VectorSubcoreMesh(core_axis_name="c", subcore_axis_name="s")

    def body(src_h, dst_h, xs_h, asrc_h, adst_h, m_h,
             acc_h, den_h,
             src_t, dst_t, m_t, zbuf,
             dstb0, dstb1, dstb2, ea0, ea1, ea2, eb0, eb1, eb2,
             pbuf0, pbuf1, pbuf2, rows0, rows1, rows2,
             gsem0, gsem1, gsem2, ssem0, ssem1, ssem2, acc_s, den_s):
        cid = lax.axis_index("c")
        sid = lax.axis_index("s")
        wid = cid * 16 + sid
        base = wid * ep
        rows_b = (rows0, rows1, rows2)
        dstb_b = (dstb0, dstb1, dstb2)
        ea_b = (ea0, ea1, ea2)
        eb_b = (eb0, eb1, eb2)
        pbuf_b = (pbuf0, pbuf1, pbuf2)
        gsem_b = (gsem0, gsem1, gsem2)
        ssem_b = (ssem0, ssem1, ssem2)
        pltpu.sync_copy(m_h.at[pl.ds(0, 16)], m_t)
        z16 = jnp.zeros((16,), F32)

        @pl.loop(0, _EB)
        def _(r):
            for k in range(8):
                rows0[r, pl.ds(k * 16, 16)] = z16

        @pl.loop(0, rows_per_tile, step=16)
        def _(i):
            zbuf[pl.ds(i, 16)] = z16

        off = sid * rows_per_tile
        for j in range(rows_per_tile // _EB):
            pltpu.sync_copy(rows0, acc_s.at[pl.ds(off + j * _EB, _EB)])
        pltpu.sync_copy(zbuf, den_s.at[pl.ds(off, rows_per_tile)])
        plsc.subcore_barrier()

        m_v = m_t[...]

        def stage_and_gather(c, b):
            cb = c * _EB
            for k in range(_EB // 16):
                dstb_b[b][pl.ds(k * 16, 16)] = dst_t[pl.ds(cb + k * 16, 16)]
            pltpu.async_copy(xs_h.at[src_t.at[pl.ds(cb, _EB)]],
                             rows_b[b], gsem_b[b])
            pltpu.async_copy(asrc_h.at[src_t.at[pl.ds(cb, _EB)]],
                             ea_b[b], gsem_b[b])
            pltpu.async_copy(adst_h.at[dstb_b[b]], eb_b[b], gsem_b[b])

        def wait_gather(b):
            pltpu.make_async_copy(xs_h.at[src_t.at[pl.ds(0, _EB)]],
                                  rows_b[b], gsem_b[b]).wait()
            pltpu.make_async_copy(asrc_h.at[src_t.at[pl.ds(0, _EB)]],
                                  ea_b[b], gsem_b[b]).wait()
            pltpu.make_async_copy(adst_h.at[dstb_b[b]], eb_b[b],
                                  gsem_b[b]).wait()

        def scale_and_scatter(b):
            wait_gather(b)
            for k in range(_EB // 16):
                ev = (ea_b[b][pl.ds(k * 16, 16)]
                      + eb_b[b][pl.ds(k * 16, 16)])
                ev = jnp.where(ev > 0, ev, ev * 0.2)
                pbuf_b[b][pl.ds(k * 16, 16)] = jnp.exp(ev - m_v)

            @pl.loop(0, _EB, step=2)
            def _(r):
                for rr in range(2):
                    pv = plsc.load_gather(
                        pbuf_b[b], [lax.broadcast(r + rr, (16,))])
                    for k in range(8):
                        rows_b[b][r + rr, pl.ds(k * 16, 16)] = (
                            rows_b[b][r + rr, pl.ds(k * 16, 16)] * pv)

            pltpu.async_copy(rows_b[b], acc_s.at[dstb_b[b]], ssem_b[b],
                             add=True)
            pltpu.async_copy(pbuf_b[b], den_s.at[dstb_b[b]], ssem_b[b],
                             add=True)

        def wait_scatter(b):
            pltpu.make_async_copy(rows_b[b], acc_s.at[dstb_b[b]],
                                  ssem_b[b]).wait()
            pltpu.make_async_copy(pbuf_b[b], den_s.at[dstb_b[b]],
                                  ssem_b[b]).wait()

        @pl.loop(0, nblk)
        def _(blk):
            bb = base + blk * eblk
            pltpu.sync_copy(src_h.at[pl.ds(bb, eblk)], src_t)
            pltpu.sync_copy(dst_h.at[pl.ds(bb, eblk)], dst_t)

            stage_and_gather(0, 0)
            stage_and_gather(1, 1)
            scale_and_scatter(0)
            stage_and_gather(2, 2)
            scale_and_scatter(1)

            @pl.loop(0, (nchunk - 4) // 3)
            def _(t):
                for u in range(3):
                    cc = 3 * t + 2 + u
                    b = (2 + u) % 3
                    nb = (b + 1) % 3
                    wait_scatter(nb)
                    stage_and_gather(cc + 1, nb)
                    scale_and_scatter(b)

            b = (nchunk - 2) % 3
            nb = (nchunk - 1) % 3
            wait_scatter(nb)
            stage_and_gather(nchunk - 1, nb)
            scale_and_scatter(b)
            scale_and_scatter(nb)
            wait_scatter((nchunk - 3) % 3)
            wait_scatter(b)
            wait_scatter(nb)

        plsc.subcore_barrier()
        for j in range(rows_per_tile // _EB):
            o = off + j * _EB
            pltpu.sync_copy(acc_s.at[pl.ds(o, _EB)],
                            acc_h.at[cid, pl.ds(o, _EB)])
        pltpu.sync_copy(den_s.at[pl.ds(off, rows_per_tile)],
                        den_h.at[cid, pl.ds(off, rows_per_tile)])

    cp = pltpu.CompilerParams()
    if "needs_layout_passes" in pltpu.CompilerParams.__dataclass_fields__:
        cp = dataclasses.replace(cp, needs_layout_passes=False)
    k = pl.kernel(
        body,
        out_type=[jax.ShapeDtypeStruct((2, _NPAD, 128), F32),
                  jax.ShapeDtypeStruct((2, _NPAD), F32)],
        mesh=mesh,
        compiler_params=cp,
        scratch_types=(
            [pltpu.VMEM((eblk,), jnp.int32),
             pltpu.VMEM((eblk,), jnp.int32),
             pltpu.VMEM((16,), F32),
             pltpu.VMEM((rows_per_tile,), F32)]
            + [pltpu.VMEM((_EB,), jnp.int32)] * 3
            + [pltpu.VMEM((_EB,), F32)] * 6
            + [pltpu.VMEM((_EB,), F32)] * 3
            + [pltpu.VMEM((_EB, 128), F32)] * 3
            + [pltpu.SemaphoreType.DMA] * 6
            + [pltpu.VMEM_SHARED((_NPAD, 128), F32),
               pltpu.VMEM_SHARED((_NPAD,), F32)]
        ),
    )
    return k(src, dst, xs, asrc, adst, m)


def kernel(x, edge_index, Ws1, Wd1, as1, ad1, b1, Wl1, bl1,
           Ws2, Wd2, as2, ad2, b2, Wl2, bl2):
    n = x.shape[0]
    src = edge_index[0]
    dst = edge_index[1]

    xs1, hlin1, asrc1, adst1, m1 = _precompute(x, Ws1, Wd1, as1, ad1, Wl1, bl1)
    acc1, den1 = _gat_sc(src, dst, xs1, asrc1.reshape(n), adst1.reshape(n),
                         m1.reshape(128))
    xs2, hlin2, asrc2, adst2, m2 = _combine_pre(
        acc1, den1.T, hlin1, b1, Ws2, Wd2, as2, ad2, Wl2, bl2)
    acc2, den2 = _gat_sc(src, dst, xs2, asrc2.reshape(n), adst2.reshape(n),
                         m2.reshape(128))
    return _final(acc2, den2.T, hlin2, b2)


# speedup vs baseline: 55.2873x; 1.1290x over previous
import dataclasses
import functools

import jax
import jax.numpy as jnp
from jax import lax
from jax.experimental import pallas as pl
from jax.experimental.pallas import tpu as pltpu
from jax.experimental.pallas import tpu_sc as plsc

F32 = jnp.float32
_BLK = 1000
_NPAD = 10240
_EB = 80


def _leaky(v):
    return jnp.where(v > 0, v, 0.2 * v)


def _layer_pre(xb, Ws, Wd, Wl, asv, adv, bl, i, nsteps,
               xs_ref, hlin_ref, asrc_ref, adst_ref, m_ref, sm):
    xs = jnp.dot(xb, Ws, preferred_element_type=F32)
    xs_ref[...] = xs
    hlin_ref[...] = jnp.dot(xb, Wl, preferred_element_type=F32) + bl
    a_s = jnp.dot(xs, asv, preferred_element_type=F32)
    asrc_ref[...] = a_s
    wda = jnp.dot(Wd, adv, preferred_element_type=F32)
    a_d = jnp.dot(xb, wda, preferred_element_type=F32)
    adst_ref[...] = a_d
    bs = jnp.max(a_s)
    bd = jnp.max(a_d)

    @pl.when(i == 0)
    def _():
        sm[0] = bs
        sm[1] = bd

    @pl.when(i != 0)
    def _():
        sm[0] = jnp.maximum(sm[0], bs)
        sm[1] = jnp.maximum(sm[1], bd)

    @pl.when(i == nsteps - 1)
    def _():
        m_ref[...] = jnp.full((1, 128), _leaky(sm[0] + sm[1]), F32)


def _pre_body(x_ref, Ws_ref, Wd_ref, Wl_ref, asv_ref, adv_ref, bl_ref,
              xs_ref, hlin_ref, asrc_ref, adst_ref, m_ref, sm):
    i = pl.program_id(0)
    _layer_pre(x_ref[...], Ws_ref[...], Wd_ref[...], Wl_ref[...],
               asv_ref[...], adv_ref[...], bl_ref[...], i, pl.num_programs(0),
               xs_ref, hlin_ref, asrc_ref, adst_ref, m_ref, sm)


def _comb_body(acc_ref, dn_ref, hlin1_ref, b_ref,
               Ws_ref, Wd_ref, Wl_ref, asv_ref, adv_ref, bl_ref,
               xs_ref, hlin_ref, asrc_ref, adst_ref, m_ref, sm):
    i = pl.program_id(0)
    a = acc_ref[0] + acc_ref[1]
    den = jnp.sum(dn_ref[...], axis=1, keepdims=True)
    h = jnp.maximum(a / (den + 1e-16) + b_ref[...] + hlin1_ref[...], 0.0)
    _layer_pre(h, Ws_ref[...], Wd_ref[...], Wl_ref[...],
               asv_ref[...], adv_ref[...], bl_ref[...], i, pl.num_programs(0),
               xs_ref, hlin_ref, asrc_ref, adst_ref, m_ref, sm)


def _final_body(acc_ref, dn_ref, hlin_ref, b_ref, out_ref):
    a = acc_ref[0] + acc_ref[1]
    den = jnp.sum(dn_ref[...], axis=1, keepdims=True)
    out_ref[...] = a / (den + 1e-16) + b_ref[...] + hlin_ref[...]


def _w_spec():
    return pl.BlockSpec((128, 128), lambda i: (0, 0))


def _layer_out(n):
    out_shape = [jax.ShapeDtypeStruct((n, 128), F32),
                 jax.ShapeDtypeStruct((n, 128), F32),
                 jax.ShapeDtypeStruct((n, 1), F32),
                 jax.ShapeDtypeStruct((n, 1), F32),
                 jax.ShapeDtypeStruct((1, 128), F32)]
    out_specs = [pl.BlockSpec((_BLK, 128), lambda i: (i, 0)),
                 pl.BlockSpec((_BLK, 128), lambda i: (i, 0)),
                 pl.BlockSpec((_BLK, 1), lambda i: (i, 0)),
                 pl.BlockSpec((_BLK, 1), lambda i: (i, 0)),
                 pl.BlockSpec((1, 128), lambda i: (0, 0))]
    return out_shape, out_specs


def _precompute(x, Ws, Wd, a_s, a_d, Wl, bl):
    n = x.shape[0]
    out_shape, out_specs = _layer_out(n)
    return pl.pallas_call(
        _pre_body,
        grid=(n // _BLK,),
        in_specs=[pl.BlockSpec((_BLK, 128), lambda i: (i, 0)),
                  _w_spec(), _w_spec(), _w_spec(),
                  pl.BlockSpec((128, 1), lambda i: (0, 0)),
                  pl.BlockSpec((128, 1), lambda i: (0, 0)),
                  pl.BlockSpec((1, 128), lambda i: (0, 0))],
        out_specs=out_specs,
        out_shape=out_shape,
        scratch_shapes=[pltpu.SMEM((2,), F32)],
    )(x, Ws, Wd, Wl, a_s.reshape(128, 1), a_d.reshape(128, 1),
      bl.reshape(1, 128))


def _combine_pre(acc, dnt, hlin1, b, Ws, Wd, a_s, a_d, Wl, bl):
    n = hlin1.shape[0]
    out_shape, out_specs = _layer_out(n)
    return pl.pallas_call(
        _comb_body,
        grid=(n // _BLK,),
        in_specs=[pl.BlockSpec((2, _BLK, 128), lambda i: (0, i, 0)),
                  pl.BlockSpec((_BLK, 2), lambda i: (i, 0)),
                  pl.BlockSpec((_BLK, 128), lambda i: (i, 0)),
                  pl.BlockSpec((1, 128), lambda i: (0, 0)),
                  _w_spec(), _w_spec(), _w_spec(),
                  pl.BlockSpec((128, 1), lambda i: (0, 0)),
                  pl.BlockSpec((128, 1), lambda i: (0, 0)),
                  pl.BlockSpec((1, 128), lambda i: (0, 0))],
        out_specs=out_specs,
        out_shape=out_shape,
        scratch_shapes=[pltpu.SMEM((2,), F32)],
    )(acc, dnt, hlin1, b.reshape(1, 128), Ws, Wd, Wl,
      a_s.reshape(128, 1), a_d.reshape(128, 1), bl.reshape(1, 128))


def _final(acc, dnt, hlin, b):
    n = hlin.shape[0]
    return pl.pallas_call(
        _final_body,
        grid=(n // _BLK,),
        in_specs=[pl.BlockSpec((2, _BLK, 128), lambda i: (0, i, 0)),
                  pl.BlockSpec((_BLK, 2), lambda i: (i, 0)),
                  pl.BlockSpec((_BLK, 128), lambda i: (i, 0)),
                  pl.BlockSpec((1, 128), lambda i: (0, 0))],
        out_specs=pl.BlockSpec((_BLK, 128), lambda i: (i, 0)),
        out_shape=jax.ShapeDtypeStruct((n, 128), F32),
    )(acc, dnt, hlin, b.reshape(1, 128))


def _gat_sc(src, dst, xs, asrc, adst, m):
    n = xs.shape[0]
    e = src.shape[0]
    ep = e // 32
    eblk = 2000
    nblk = ep // eblk
    nchunk = eblk // _EB
    assert e % 32 == 0 and ep % eblk == 0 and eblk % _EB == 0
    rows_per_tile = _NPAD // 16

    mesh = plsc.VectorSubcoreMesh(core_axis_name="c", subcore_axis_name="s")

    acc_rows = n // 16

    def body(src_h, dst_h, xs_h, asrc_h, adst_h, m_h,
             acc_h, den_h,
             src_t, dst_t, m_t, zbuf,
             dstb0, dstb1, dstb2, dstb3, ea0, ea1, ea2, ea3,
             eb0, eb1, eb2, eb3, pbuf0, pbuf1, pbuf2, pbuf3,
             rows0, rows1, rows2, rows3,
             gsem0, gsem1, gsem2, gsem3, ssem0, ssem1, ssem2, ssem3,
             acc_s, den_s):
        cid = lax.axis_index("c")
        sid = lax.axis_index("s")
        wid = cid * 16 + sid
        base = wid * ep
        rows_b = (rows0, rows1, rows2, rows3)
        dstb_b = (dstb0, dstb1, dstb2, dstb3)
        ea_b = (ea0, ea1, ea2, ea3)
        eb_b = (eb0, eb1, eb2, eb3)
        pbuf_b = (pbuf0, pbuf1, pbuf2, pbuf3)
        gsem_b = (gsem0, gsem1, gsem2, gsem3)
        ssem_b = (ssem0, ssem1, ssem2, ssem3)
        pltpu.sync_copy(m_h.at[pl.ds(0, 16)], m_t)
        z16 = jnp.zeros((16,), F32)

        @pl.loop(0, _EB)
        def _(r):
            for k in range(8):
                rows0[r, pl.ds(k * 16, 16)] = z16

        @pl.loop(0, rows_per_tile, step=16)
        def _(i):
            zbuf[pl.ds(i, 16)] = z16

        off = sid * rows_per_tile
        for j in range(rows_per_tile // _EB):
            pltpu.sync_copy(rows0, acc_s.at[pl.ds(off + j * _EB, _EB)])
        pltpu.sync_copy(zbuf, den_s.at[pl.ds(off, rows_per_tile)])
        plsc.subcore_barrier()

        m_v = m_t[...]

        def stage_and_gather(c, b):
            cb = c * _EB
            for k in range(_EB // 16):
                dstb_b[b][pl.ds(k * 16, 16)] = dst_t[pl.ds(cb + k * 16, 16)]
            pltpu.async_copy(xs_h.at[src_t.at[pl.ds(cb, _EB)]],
                             rows_b[b], gsem_b[b])
            pltpu.async_copy(asrc_h.at[src_t.at[pl.ds(cb, _EB)]],
                             ea_b[b], gsem_b[b])
            pltpu.async_copy(adst_h.at[dstb_b[b]], eb_b[b], gsem_b[b])

        def wait_gather(b):
            pltpu.make_async_copy(xs_h.at[src_t.at[pl.ds(0, _EB)]],
                                  rows_b[b], gsem_b[b]).wait()
            pltpu.make_async_copy(asrc_h.at[src_t.at[pl.ds(0, _EB)]],
                                  ea_b[b], gsem_b[b]).wait()
            pltpu.make_async_copy(adst_h.at[dstb_b[b]], eb_b[b],
                                  gsem_b[b]).wait()

        def scale_and_scatter(b):
            wait_gather(b)
            for k in range(_EB // 16):
                ev = (ea_b[b][pl.ds(k * 16, 16)]
                      + eb_b[b][pl.ds(k * 16, 16)])
                ev = jnp.where(ev > 0, ev, ev * 0.2)
                pbuf_b[b][pl.ds(k * 16, 16)] = jnp.exp(ev - m_v)

            @pl.loop(0, _EB, step=2)
            def _(r):
                for rr in range(2):
                    pv = plsc.load_gather(
                        pbuf_b[b], [lax.broadcast(r + rr, (16,))])
                    for k in range(8):
                        rows_b[b][r + rr, pl.ds(k * 16, 16)] = (
                            rows_b[b][r + rr, pl.ds(k * 16, 16)] * pv)

            pltpu.async_copy(rows_b[b], acc_s.at[dstb_b[b]], ssem_b[b],
                             add=True)
            pltpu.async_copy(pbuf_b[b], den_s.at[dstb_b[b]], ssem_b[b],
                             add=True)

        def wait_scatter(b):
            pltpu.make_async_copy(rows_b[b], acc_s.at[dstb_b[b]],
                                  ssem_b[b]).wait()
            pltpu.make_async_copy(pbuf_b[b], den_s.at[dstb_b[b]],
                                  ssem_b[b]).wait()

        @pl.loop(0, nblk)
        def _(blk):
            bb = base + blk * eblk
            pltpu.sync_copy(src_h.at[pl.ds(bb, eblk)], src_t)
            pltpu.sync_copy(dst_h.at[pl.ds(bb, eblk)], dst_t)

            stage_and_gather(0, 0)
            stage_and_gather(1, 1)
            stage_and_gather(2, 2)
            scale_and_scatter(0)
            stage_and_gather(3, 3)
            scale_and_scatter(1)

            @pl.loop(0, (nchunk - 5) // 4)
            def _(t):
                for u in range(4):
                    cc = 4 * t + 2 + u
                    b = (2 + u) % 4
                    nb = (b + 2) % 4
                    wait_scatter(nb)
                    stage_and_gather(cc + 2, nb)
                    scale_and_scatter(b)

            b = (nchunk - 3) % 4
            wait_scatter((b + 2) % 4)
            stage_and_gather(nchunk - 1, (b + 2) % 4)
            scale_and_scatter(b)
            scale_and_scatter((b + 1) % 4)
            scale_and_scatter((b + 2) % 4)
            wait_scatter((b + 3) % 4)
            wait_scatter(b)
            wait_scatter((b + 1) % 4)
            wait_scatter((b + 2) % 4)

        plsc.subcore_barrier()
        for j in range(rows_per_tile // _EB):
            o = off + j * _EB
            pltpu.sync_copy(acc_s.at[pl.ds(o, _EB)],
                            acc_h.at[cid, pl.ds(o, _EB)])
        pltpu.sync_copy(den_s.at[pl.ds(off, rows_per_tile)],
                        den_h.at[cid, pl.ds(off, rows_per_tile)])

    cp = pltpu.CompilerParams()
    if "needs_layout_passes" in pltpu.CompilerParams.__dataclass_fields__:
        cp = dataclasses.replace(cp, needs_layout_passes=False)
    k = pl.kernel(
        body,
        out_type=[jax.ShapeDtypeStruct((2, _NPAD, 128), F32),
                  jax.ShapeDtypeStruct((2, _NPAD), F32)],
        mesh=mesh,
        compiler_params=cp,
        scratch_types=(
            [pltpu.VMEM((eblk,), jnp.int32),
             pltpu.VMEM((eblk,), jnp.int32),
             pltpu.VMEM((16,), F32),
             pltpu.VMEM((rows_per_tile,), F32)]
            + [pltpu.VMEM((_EB,), jnp.int32)] * 4
            + [pltpu.VMEM((_EB,), F32)] * 8
            + [pltpu.VMEM((_EB,), F32)] * 4
            + [pltpu.VMEM((_EB, 128), F32)] * 4
            + [pltpu.SemaphoreType.DMA] * 8
            + [pltpu.VMEM_SHARED((_NPAD, 128), F32),
               pltpu.VMEM_SHARED((_NPAD,), F32)]
        ),
    )
    return k(src, dst, xs, asrc, adst, m)


def kernel(x, edge_index, Ws1, Wd1, as1, ad1, b1, Wl1, bl1,
           Ws2, Wd2, as2, ad2, b2, Wl2, bl2):
    n = x.shape[0]
    src = edge_index[0]
    dst = edge_index[1]

    xs1, hlin1, asrc1, adst1, m1 = _precompute(x, Ws1, Wd1, as1, ad1, Wl1, bl1)
    acc1, den1 = _gat_sc(src, dst, xs1, asrc1.reshape(n), adst1.reshape(n),
                         m1.reshape(128))
    xs2, hlin2, asrc2, adst2, m2 = _combine_pre(
        acc1, den1.T, hlin1, b1, Ws2, Wd2, as2, ad2, Wl2, bl2)
    acc2, den2 = _gat_sc(src, dst, xs2, asrc2.reshape(n), adst2.reshape(n),
                         m2.reshape(128))
    return _final(acc2, den2.T, hlin2, b2)


# speedup vs baseline: 56.5750x; 1.0233x over previous
import dataclasses
import functools

import jax
import jax.numpy as jnp
from jax import lax
from jax.experimental import pallas as pl
from jax.experimental.pallas import tpu as pltpu
from jax.experimental.pallas import tpu_sc as plsc

F32 = jnp.float32
_BLK = 1000
_NPAD = 10240
_EB = 80


def _leaky(v):
    return jnp.where(v > 0, v, 0.2 * v)


def _layer_pre(xb, Ws, Wd, Wl, asv, adv, bl, i, nsteps,
               xs_ref, hlin_ref, asrc_ref, adst_ref, m_ref, sm):
    xs = jnp.dot(xb, Ws, preferred_element_type=F32)
    xs_ref[...] = xs
    hlin_ref[...] = jnp.dot(xb, Wl, preferred_element_type=F32) + bl
    a_s = jnp.dot(xs, asv, preferred_element_type=F32)
    asrc_ref[...] = a_s
    wda = jnp.dot(Wd, adv, preferred_element_type=F32)
    a_d = jnp.dot(xb, wda, preferred_element_type=F32)
    adst_ref[...] = a_d
    bs = jnp.max(a_s)
    bd = jnp.max(a_d)

    @pl.when(i == 0)
    def _():
        sm[0] = bs
        sm[1] = bd

    @pl.when(i != 0)
    def _():
        sm[0] = jnp.maximum(sm[0], bs)
        sm[1] = jnp.maximum(sm[1], bd)

    @pl.when(i == nsteps - 1)
    def _():
        m_ref[...] = jnp.full((1, 128), _leaky(sm[0] + sm[1]), F32)


def _pre_body(x_ref, Ws_ref, Wd_ref, Wl_ref, asv_ref, adv_ref, bl_ref,
              xs_ref, hlin_ref, asrc_ref, adst_ref, m_ref, sm):
    i = pl.program_id(0)
    _layer_pre(x_ref[...], Ws_ref[...], Wd_ref[...], Wl_ref[...],
               asv_ref[...], adv_ref[...], bl_ref[...], i, pl.num_programs(0),
               xs_ref, hlin_ref, asrc_ref, adst_ref, m_ref, sm)


def _comb_body(acc_ref, dn_ref, hlin1_ref, b_ref,
               Ws_ref, Wd_ref, Wl_ref, asv_ref, adv_ref, bl_ref,
               xs_ref, hlin_ref, asrc_ref, adst_ref, m_ref, sm):
    i = pl.program_id(0)
    a = acc_ref[0] + acc_ref[1]
    den = jnp.sum(dn_ref[...], axis=1, keepdims=True)
    h = jnp.maximum(a / (den + 1e-16) + b_ref[...] + hlin1_ref[...], 0.0)
    _layer_pre(h, Ws_ref[...], Wd_ref[...], Wl_ref[...],
               asv_ref[...], adv_ref[...], bl_ref[...], i, pl.num_programs(0),
               xs_ref, hlin_ref, asrc_ref, adst_ref, m_ref, sm)


def _final_body(acc_ref, dn_ref, hlin_ref, b_ref, out_ref):
    a = acc_ref[0] + acc_ref[1]
    den = jnp.sum(dn_ref[...], axis=1, keepdims=True)
    out_ref[...] = a / (den + 1e-16) + b_ref[...] + hlin_ref[...]


def _w_spec():
    return pl.BlockSpec((128, 128), lambda i: (0, 0))


def _layer_out(n):
    out_shape = [jax.ShapeDtypeStruct((n, 128), F32),
                 jax.ShapeDtypeStruct((n, 128), F32),
                 jax.ShapeDtypeStruct((n, 1), F32),
                 jax.ShapeDtypeStruct((n, 1), F32),
                 jax.ShapeDtypeStruct((1, 128), F32)]
    out_specs = [pl.BlockSpec((_BLK, 128), lambda i: (i, 0)),
                 pl.BlockSpec((_BLK, 128), lambda i: (i, 0)),
                 pl.BlockSpec((_BLK, 1), lambda i: (i, 0)),
                 pl.BlockSpec((_BLK, 1), lambda i: (i, 0)),
                 pl.BlockSpec((1, 128), lambda i: (0, 0))]
    return out_shape, out_specs


def _precompute(x, Ws, Wd, a_s, a_d, Wl, bl):
    n = x.shape[0]
    out_shape, out_specs = _layer_out(n)
    return pl.pallas_call(
        _pre_body,
        grid=(n // _BLK,),
        in_specs=[pl.BlockSpec((_BLK, 128), lambda i: (i, 0)),
                  _w_spec(), _w_spec(), _w_spec(),
                  pl.BlockSpec((128, 1), lambda i: (0, 0)),
                  pl.BlockSpec((128, 1), lambda i: (0, 0)),
                  pl.BlockSpec((1, 128), lambda i: (0, 0))],
        out_specs=out_specs,
        out_shape=out_shape,
        scratch_shapes=[pltpu.SMEM((2,), F32)],
    )(x, Ws, Wd, Wl, a_s.reshape(128, 1), a_d.reshape(128, 1),
      bl.reshape(1, 128))


def _combine_pre(acc, dnt, hlin1, b, Ws, Wd, a_s, a_d, Wl, bl):
    n = hlin1.shape[0]
    out_shape, out_specs = _layer_out(n)
    return pl.pallas_call(
        _comb_body,
        grid=(n // _BLK,),
        in_specs=[pl.BlockSpec((2, _BLK, 128), lambda i: (0, i, 0)),
                  pl.BlockSpec((_BLK, 2), lambda i: (i, 0)),
                  pl.BlockSpec((_BLK, 128), lambda i: (i, 0)),
                  pl.BlockSpec((1, 128), lambda i: (0, 0)),
                  _w_spec(), _w_spec(), _w_spec(),
                  pl.BlockSpec((128, 1), lambda i: (0, 0)),
                  pl.BlockSpec((128, 1), lambda i: (0, 0)),
                  pl.BlockSpec((1, 128), lambda i: (0, 0))],
        out_specs=out_specs,
        out_shape=out_shape,
        scratch_shapes=[pltpu.SMEM((2,), F32)],
    )(acc, dnt, hlin1, b.reshape(1, 128), Ws, Wd, Wl,
      a_s.reshape(128, 1), a_d.reshape(128, 1), bl.reshape(1, 128))


def _final(acc, dnt, hlin, b):
    n = hlin.shape[0]
    return pl.pallas_call(
        _final_body,
        grid=(n // _BLK,),
        in_specs=[pl.BlockSpec((2, _BLK, 128), lambda i: (0, i, 0)),
                  pl.BlockSpec((_BLK, 2), lambda i: (i, 0)),
                  pl.BlockSpec((_BLK, 128), lambda i: (i, 0)),
                  pl.BlockSpec((1, 128), lambda i: (0, 0))],
        out_specs=pl.BlockSpec((_BLK, 128), lambda i: (i, 0)),
        out_shape=jax.ShapeDtypeStruct((n, 128), F32),
    )(acc, dnt, hlin, b.reshape(1, 128))


def _gat_sc(src, dst, xs, asrc, adst, m):
    n = xs.shape[0]
    e = src.shape[0]
    ep = e // 32
    eblk = 2000
    nblk = ep // eblk
    nchunk = eblk // _EB
    assert e % 32 == 0 and ep % eblk == 0 and eblk % _EB == 0
    rows_per_tile = _NPAD // 16

    mesh = plsc.VectorSubcoreMesh(core_axis_name="c", subcore_axis_name="s")

    acc_rows = n // 16

    def body(src_h, dst_h, xs_h, asrc_h, adst_h, m_h,
             acc_h, den_h,
             src_t, dst_t, m_t, zbuf,
             dstb0, dstb1, dstb2, dstb3, ea0, ea1, ea2, ea3,
             eb0, eb1, eb2, eb3, pbuf0, pbuf1, pbuf2, pbuf3,
             rows0, rows1, rows2, rows3,
             gsem0, gsem1, gsem2, gsem3, ssem0, ssem1, ssem2, ssem3,
             acc_s, den_s):
        cid = lax.axis_index("c")
        sid = lax.axis_index("s")
        wid = cid * 16 + sid
        base = wid * ep
        rows_b = (rows0, rows1, rows2, rows3)
        dstb_b = (dstb0, dstb1, dstb2, dstb3)
        ea_b = (ea0, ea1, ea2, ea3)
        eb_b = (eb0, eb1, eb2, eb3)
        pbuf_b = (pbuf0, pbuf1, pbuf2, pbuf3)
        gsem_b = (gsem0, gsem1, gsem2, gsem3)
        ssem_b = (ssem0, ssem1, ssem2, ssem3)
        pltpu.sync_copy(m_h.at[pl.ds(0, 16)], m_t)
        z16 = jnp.zeros((16,), F32)

        @pl.loop(0, _EB)
        def _(r):
            for k in range(8):
                rows0[r, pl.ds(k * 16, 16)] = z16

        @pl.loop(0, rows_per_tile, step=16)
        def _(i):
            zbuf[pl.ds(i, 16)] = z16

        off = sid * rows_per_tile
        for j in range(rows_per_tile // _EB):
            pltpu.sync_copy(rows0, acc_s.at[pl.ds(off + j * _EB, _EB)])
        pltpu.sync_copy(zbuf, den_s.at[pl.ds(off, rows_per_tile)])
        plsc.subcore_barrier()

        m_v = m_t[...]

        def stage_and_gather(c, b):
            cb = c * _EB
            for k in range(_EB // 16):
                dstb_b[b][pl.ds(k * 16, 16)] = dst_t[pl.ds(cb + k * 16, 16)]
            pltpu.async_copy(xs_h.at[src_t.at[pl.ds(cb, _EB)]],
                             rows_b[b], gsem_b[b])
            pltpu.async_copy(asrc_h.at[src_t.at[pl.ds(cb, _EB)]],
                             ea_b[b], gsem_b[b])
            pltpu.async_copy(adst_h.at[dstb_b[b]], eb_b[b], gsem_b[b])

        def wait_gather(b):
            pltpu.make_async_copy(xs_h.at[src_t.at[pl.ds(0, _EB)]],
                                  rows_b[b], gsem_b[b]).wait()
            pltpu.make_async_copy(asrc_h.at[src_t.at[pl.ds(0, _EB)]],
                                  ea_b[b], gsem_b[b]).wait()
            pltpu.make_async_copy(adst_h.at[dstb_b[b]], eb_b[b],
                                  gsem_b[b]).wait()

        def scale_and_scatter(b):
            wait_gather(b)
            for k in range(_EB // 16):
                ev = (ea_b[b][pl.ds(k * 16, 16)]
                      + eb_b[b][pl.ds(k * 16, 16)])
                ev = jnp.where(ev > 0, ev, ev * 0.2)
                pbuf_b[b][pl.ds(k * 16, 16)] = jnp.exp(ev - m_v)

            @pl.loop(0, _EB, step=16)
            def _(r0):
                p16 = pbuf_b[b][pl.ds(r0, 16)]
                for j in range(16):
                    pv = lax.broadcast(p16[j], (16,))
                    for k in range(8):
                        rows_b[b][r0 + j, pl.ds(k * 16, 16)] = (
                            rows_b[b][r0 + j, pl.ds(k * 16, 16)] * pv)

            pltpu.async_copy(rows_b[b], acc_s.at[dstb_b[b]], ssem_b[b],
                             add=True)
            pltpu.async_copy(pbuf_b[b], den_s.at[dstb_b[b]], ssem_b[b],
                             add=True)

        def wait_scatter(b):
            pltpu.make_async_copy(rows_b[b], acc_s.at[dstb_b[b]],
                                  ssem_b[b]).wait()
            pltpu.make_async_copy(pbuf_b[b], den_s.at[dstb_b[b]],
                                  ssem_b[b]).wait()

        @pl.loop(0, nblk)
        def _(blk):
            bb = base + blk * eblk
            pltpu.sync_copy(src_h.at[pl.ds(bb, eblk)], src_t)
            pltpu.sync_copy(dst_h.at[pl.ds(bb, eblk)], dst_t)

            stage_and_gather(0, 0)
            stage_and_gather(1, 1)
            stage_and_gather(2, 2)
            scale_and_scatter(0)
            stage_and_gather(3, 3)
            scale_and_scatter(1)

            @pl.loop(0, (nchunk - 5) // 4)
            def _(t):
                for u in range(4):
                    cc = 4 * t + 2 + u
                    b = (2 + u) % 4
                    nb = (b + 2) % 4
                    wait_scatter(nb)
                    stage_and_gather(cc + 2, nb)
                    scale_and_scatter(b)

            b = (nchunk - 3) % 4
            wait_scatter((b + 2) % 4)
            stage_and_gather(nchunk - 1, (b + 2) % 4)
            scale_and_scatter(b)
            scale_and_scatter((b + 1) % 4)
            scale_and_scatter((b + 2) % 4)
            wait_scatter((b + 3) % 4)
            wait_scatter(b)
            wait_scatter((b + 1) % 4)
            wait_scatter((b + 2) % 4)

        plsc.subcore_barrier()
        for j in range(rows_per_tile // _EB):
            o = off + j * _EB
            pltpu.sync_copy(acc_s.at[pl.ds(o, _EB)],
                            acc_h.at[cid, pl.ds(o, _EB)])
        pltpu.sync_copy(den_s.at[pl.ds(off, rows_per_tile)],
                        den_h.at[cid, pl.ds(off, rows_per_tile)])

    cp = pltpu.CompilerParams()
    if "needs_layout_passes" in pltpu.CompilerParams.__dataclass_fields__:
        cp = dataclasses.replace(cp, needs_layout_passes=False)
    k = pl.kernel(
        body,
        out_type=[jax.ShapeDtypeStruct((2, _NPAD, 128), F32),
                  jax.ShapeDtypeStruct((2, _NPAD), F32)],
        mesh=mesh,
        compiler_params=cp,
        scratch_types=(
            [pltpu.VMEM((eblk,), jnp.int32),
             pltpu.VMEM((eblk,), jnp.int32),
             pltpu.VMEM((16,), F32),
             pltpu.VMEM((rows_per_tile,), F32)]
            + [pltpu.VMEM((_EB,), jnp.int32)] * 4
            + [pltpu.VMEM((_EB,), F32)] * 8
            + [pltpu.VMEM((_EB,), F32)] * 4
            + [pltpu.VMEM((_EB, 128), F32)] * 4
            + [pltpu.SemaphoreType.DMA] * 8
            + [pltpu.VMEM_SHARED((_NPAD, 128), F32),
               pltpu.VMEM_SHARED((_NPAD,), F32)]
        ),
    )
    return k(src, dst, xs, asrc, adst, m)


def kernel(x, edge_index, Ws1, Wd1, as1, ad1, b1, Wl1, bl1,
           Ws2, Wd2, as2, ad2, b2, Wl2, bl2):
    n = x.shape[0]
    src = edge_index[0]
    dst = edge_index[1]

    xs1, hlin1, asrc1, adst1, m1 = _precompute(x, Ws1, Wd1, as1, ad1, Wl1, bl1)
    acc1, den1 = _gat_sc(src, dst, xs1, asrc1.reshape(n), adst1.reshape(n),
                         m1.reshape(128))
    xs2, hlin2, asrc2, adst2, m2 = _combine_pre(
        acc1, den1.T, hlin1, b1, Ws2, Wd2, as2, ad2, Wl2, bl2)
    acc2, den2 = _gat_sc(src, dst, xs2, asrc2.reshape(n), adst2.reshape(n),
                         m2.reshape(128))
    return _final(acc2, den2.T, hlin2, b2)
